# baseline (speedup 1.0000x reference)
import functools

import jax
import jax.numpy as jnp
from jax import lax
from jax.experimental import pallas as pl
from jax.experimental.pallas import tpu as pltpu
from jax.experimental.pallas import tpu_sc as plsc

DIM = 1024
E = 8
HID = 2048
N = 2048
EP = 128
T = 256
G = 16
ROWS = G * T

NC = 2
NS = 16
NW = NC * NS
L = 16


def _gate_body(x_ref, w_ref, b_ref, idx_ref, val_ref):
    s = jnp.dot(x_ref[...], w_ref[...], preferred_element_type=jnp.float32)
    s = s + b_ref[...]
    m = jnp.max(s, axis=-1, keepdims=True)
    lane = lax.broadcasted_iota(jnp.int32, s.shape, 1)
    cand = jnp.where(s >= m, lane, EP)
    idx_ref[...] = jnp.min(cand, axis=-1, keepdims=True)
    val_ref[...] = m


def _gate(x2, gate_W, gate_b):
    wp = jnp.zeros((DIM, EP), jnp.float32).at[:, :E].set(gate_W)
    bp = jnp.full((1, EP), -1e30, jnp.float32).at[0, :E].set(gate_b)
    bt = 256
    idx, val = pl.pallas_call(
        _gate_body,
        grid=(N // bt,),
        in_specs=[
            pl.BlockSpec((bt, DIM), lambda g: (g, 0)),
            pl.BlockSpec((DIM, EP), lambda g: (0, 0)),
            pl.BlockSpec((1, EP), lambda g: (0, 0)),
        ],
        out_specs=[
            pl.BlockSpec((bt, 1), lambda g: (g, 0)),
            pl.BlockSpec((bt, 1), lambda g: (g, 0)),
        ],
        out_shape=[
            jax.ShapeDtypeStruct((N, 1), jnp.int32),
            jax.ShapeDtypeStruct((N, 1), jnp.float32),
        ],
    )(x2, wp, bp)
    return idx.reshape(N), val.reshape(N)


@functools.partial(
    pl.kernel,
    out_type=jax.ShapeDtypeStruct((ROWS, DIM), jnp.float32),
    mesh=plsc.VectorSubcoreMesh(core_axis_name="c", subcore_axis_name="s"),
    scratch_types=[
        pltpu.VMEM((ROWS // NW // 32, 32), jnp.int32),
        pltpu.VMEM((32, DIM), jnp.float32),
        pltpu.VMEM((32, DIM), jnp.float32),
        pltpu.SemaphoreType.DMA,
        pltpu.SemaphoreType.DMA,
    ],
)
def _dispatch(x_hbm, src_hbm, xpad_hbm, idx_v, buf0, buf1, sem0, sem1):
    wid = lax.axis_index("s") * NC + lax.axis_index("c")
    rpw = ROWS // NW
    base = wid * rpw
    nchunk = rpw // 32
    for c in range(nchunk):
        pltpu.sync_copy(src_hbm.at[pl.ds(base + 32 * c, 32)], idx_v.at[c])
    bufs = (buf0, buf1)
    sems = (sem0, sem1)
    cps = [None, None]
    cps[0] = pltpu.async_copy(x_hbm.at[idx_v.at[0]], bufs[0], sems[0])
    for c in range(nchunk):
        if c + 1 < nchunk:
            cps[(c + 1) % 2] = pltpu.async_copy(
                x_hbm.at[idx_v.at[c + 1]], bufs[(c + 1) % 2], sems[(c + 1) % 2])
        cps[c % 2].wait()
        pltpu.sync_copy(bufs[c % 2], xpad_hbm.at[pl.ds(base + 32 * c, 32)])


@functools.partial(
    pl.kernel,
    out_type=jax.ShapeDtypeStruct((N, DIM), jnp.float32),
    mesh=plsc.VectorSubcoreMesh(core_axis_name="c", subcore_axis_name="s"),
    scratch_types=[
        pltpu.VMEM((N // NW // 32, 32), jnp.int32),
        pltpu.VMEM((32, DIM), jnp.float32),
        pltpu.VMEM((32, DIM), jnp.float32),
        pltpu.SemaphoreType.DMA,
        pltpu.SemaphoreType.DMA,
    ],
)
def _combine(ypad_hbm, dst_hbm, out_hbm, idx_v, buf0, buf1, sem0, sem1):
    wid = lax.axis_index("s") * NC + lax.axis_index("c")
    rpw = N // NW
    base = wid * rpw
    nchunk = rpw // 32
    for c in range(nchunk):
        pltpu.sync_copy(dst_hbm.at[pl.ds(base + 32 * c, 32)], idx_v.at[c])
    bufs = (buf0, buf1)
    sems = (sem0, sem1)
    cps = [None, None]
    cps[0] = pltpu.async_copy(ypad_hbm.at[idx_v.at[0]], bufs[0], sems[0])
    for c in range(nchunk):
        if c + 1 < nchunk:
            cps[(c + 1) % 2] = pltpu.async_copy(
                ypad_hbm.at[idx_v.at[c + 1]], bufs[(c + 1) % 2],
                sems[(c + 1) % 2])
        cps[c % 2].wait()
        pltpu.sync_copy(bufs[c % 2], out_hbm.at[pl.ds(base + 32 * c, 32)])


def _ffn_body(te_ref, xp_ref, w1_ref, b1_ref, w2_ref, b2_ref, sc_ref, out_ref):
    xb = xp_ref[...].astype(jnp.bfloat16)
    h = jnp.dot(xb, w1_ref[0].astype(jnp.bfloat16),
                preferred_element_type=jnp.float32)
    h = h + b1_ref[0]
    h = 0.5 * h * (1.0 + lax.erf(h * 0.7071067811865476))
    y = jnp.dot(h.astype(jnp.bfloat16), w2_ref[0].astype(jnp.bfloat16),
                preferred_element_type=jnp.float32)
    out_ref[...] = (y + b2_ref[0]) * sc_ref[...]


def _ffn(tile_expert, x_pad, W1, b1, W2, b2, score_pad):
    return pl.pallas_call(
        _ffn_body,
        grid_spec=pltpu.PrefetchScalarGridSpec(
            num_scalar_prefetch=1,
            grid=(G,),
            in_specs=[
                pl.BlockSpec((T, DIM), lambda g, te: (g, 0)),
                pl.BlockSpec((1, DIM, HID), lambda g, te: (te[g], 0, 0)),
                pl.BlockSpec((1, 1, HID), lambda g, te: (te[g], 0, 0)),
                pl.BlockSpec((1, HID, DIM), lambda g, te: (te[g], 0, 0)),
                pl.BlockSpec((1, 1, DIM), lambda g, te: (te[g], 0, 0)),
                pl.BlockSpec((T, 1), lambda g, te: (g, 0)),
            ],
            out_specs=pl.BlockSpec((T, DIM), lambda g, te: (g, 0)),
        ),
        out_shape=jax.ShapeDtypeStruct((ROWS, DIM), jnp.float32),
        compiler_params=pltpu.CompilerParams(
            dimension_semantics=("arbitrary",),
            vmem_limit_bytes=100 * 1024 * 1024,
        ),
    )(tile_expert, x_pad, W1, b1.reshape(E, 1, HID), W2,
      b2.reshape(E, 1, DIM), score_pad)


def kernel(x, gate_W, gate_b, W1, b1, W2, b2):
    x2 = x.reshape(N, DIM)
    idx, score = _gate(x2, gate_W, gate_b)

    oh = (idx[:, None] == jnp.arange(E, dtype=jnp.int32)[None, :]).astype(jnp.int32)
    counts = oh.sum(axis=0)
    rank = jnp.take_along_axis(jnp.cumsum(oh, axis=0), idx[:, None], axis=1)[:, 0] - 1
    ntiles = (counts + T - 1) // T
    tile_off = jnp.concatenate([jnp.zeros((1,), jnp.int32),
                                jnp.cumsum(ntiles).astype(jnp.int32)])
    dst = tile_off[idx] * T + rank
    src_pad = jnp.zeros((ROWS,), jnp.int32).at[dst].set(
        jnp.arange(N, dtype=jnp.int32))
    g_ids = jnp.arange(G, dtype=jnp.int32)
    te = jnp.minimum(
        jnp.searchsorted(tile_off[1:], g_ids, side="right").astype(jnp.int32),
        E - 1)

    x_pad = _dispatch(x2, src_pad)
    score_pad = jnp.zeros((ROWS,), jnp.float32).at[dst].set(score)

    y_pad = _ffn(te, x_pad, W1, b1, W2, b2, score_pad.reshape(ROWS, 1))

    out = _combine(y_pad, dst)
    return out.reshape(1, N, DIM)


# speedup vs baseline: 1.5827x; 1.5827x over previous
import functools

import jax
import jax.numpy as jnp
from jax import lax
from jax.experimental import pallas as pl
from jax.experimental.pallas import tpu as pltpu
from jax.experimental.pallas import tpu_sc as plsc

DIM = 1024
E = 8
HID = 2048
N = 2048
EP = 128
T = 256
G = 16
ROWS = G * T

NC = 2
NS = 16
NW = NC * NS
L = 16


def _gate_body(x_ref, w_ref, b_ref, idx_ref, val_ref):
    s = jnp.dot(x_ref[...], w_ref[...], preferred_element_type=jnp.float32)
    s = s + b_ref[...]
    m = jnp.max(s, axis=-1, keepdims=True)
    lane = lax.broadcasted_iota(jnp.int32, s.shape, 1)
    cand = jnp.where(s >= m, lane, EP)
    idx_ref[...] = jnp.min(cand, axis=-1, keepdims=True)
    val_ref[...] = m


def _gate(x2, gate_W, gate_b):
    wp = jnp.zeros((DIM, EP), jnp.float32).at[:, :E].set(gate_W)
    bp = jnp.full((1, EP), -1e30, jnp.float32).at[0, :E].set(gate_b)
    bt = 256
    idx, val = pl.pallas_call(
        _gate_body,
        grid=(N // bt,),
        in_specs=[
            pl.BlockSpec((bt, DIM), lambda g: (g, 0)),
            pl.BlockSpec((DIM, EP), lambda g: (0, 0)),
            pl.BlockSpec((1, EP), lambda g: (0, 0)),
        ],
        out_specs=[
            pl.BlockSpec((bt, 1), lambda g: (g, 0)),
            pl.BlockSpec((bt, 1), lambda g: (g, 0)),
        ],
        out_shape=[
            jax.ShapeDtypeStruct((N, 1), jnp.int32),
            jax.ShapeDtypeStruct((N, 1), jnp.float32),
        ],
    )(x2, wp, bp)
    return idx.reshape(N), val.reshape(N)


@functools.partial(
    pl.kernel,
    out_type=jax.ShapeDtypeStruct((ROWS, DIM), jnp.float32),
    mesh=plsc.VectorSubcoreMesh(core_axis_name="c", subcore_axis_name="s"),
    scratch_types=[
        pltpu.VMEM((ROWS // NW // 32, 32), jnp.int32),
        pltpu.VMEM((32, DIM), jnp.float32),
        pltpu.VMEM((32, DIM), jnp.float32),
        pltpu.SemaphoreType.DMA,
        pltpu.SemaphoreType.DMA,
    ],
)
def _dispatch(x_hbm, src_hbm, xpad_hbm, idx_v, buf0, buf1, sem0, sem1):
    wid = lax.axis_index("s") * NC + lax.axis_index("c")
    rpw = ROWS // NW
    base = wid * rpw
    nchunk = rpw // 32
    for c in range(nchunk):
        pltpu.sync_copy(src_hbm.at[pl.ds(base + 32 * c, 32)], idx_v.at[c])
    bufs = (buf0, buf1)
    sems = (sem0, sem1)
    cps = [None, None]
    cps[0] = pltpu.async_copy(x_hbm.at[idx_v.at[0]], bufs[0], sems[0])
    for c in range(nchunk):
        if c + 1 < nchunk:
            cps[(c + 1) % 2] = pltpu.async_copy(
                x_hbm.at[idx_v.at[c + 1]], bufs[(c + 1) % 2], sems[(c + 1) % 2])
        cps[c % 2].wait()
        pltpu.sync_copy(bufs[c % 2], xpad_hbm.at[pl.ds(base + 32 * c, 32)])


@functools.partial(
    pl.kernel,
    out_type=jax.ShapeDtypeStruct((N, DIM), jnp.float32),
    mesh=plsc.VectorSubcoreMesh(core_axis_name="c", subcore_axis_name="s"),
    scratch_types=[
        pltpu.VMEM((N // NW // 32, 32), jnp.int32),
        pltpu.VMEM((32, DIM), jnp.float32),
        pltpu.VMEM((32, DIM), jnp.float32),
        pltpu.SemaphoreType.DMA,
        pltpu.SemaphoreType.DMA,
    ],
)
def _combine(ypad_hbm, dst_hbm, out_hbm, idx_v, buf0, buf1, sem0, sem1):
    wid = lax.axis_index("s") * NC + lax.axis_index("c")
    rpw = N // NW
    base = wid * rpw
    nchunk = rpw // 32
    for c in range(nchunk):
        pltpu.sync_copy(dst_hbm.at[pl.ds(base + 32 * c, 32)], idx_v.at[c])
    bufs = (buf0, buf1)
    sems = (sem0, sem1)
    cps = [None, None]
    cps[0] = pltpu.async_copy(ypad_hbm.at[idx_v.at[0]], bufs[0], sems[0])
    for c in range(nchunk):
        if c + 1 < nchunk:
            cps[(c + 1) % 2] = pltpu.async_copy(
                ypad_hbm.at[idx_v.at[c + 1]], bufs[(c + 1) % 2],
                sems[(c + 1) % 2])
        cps[c % 2].wait()
        pltpu.sync_copy(bufs[c % 2], out_hbm.at[pl.ds(base + 32 * c, 32)])


def _ffn_body(te_ref, xp_ref, w1_ref, b1_ref, w2_ref, b2_ref, sc_ref, out_ref):
    xb = xp_ref[...].astype(jnp.bfloat16)
    h = jnp.dot(xb, w1_ref[0].astype(jnp.bfloat16),
                preferred_element_type=jnp.float32)
    h = h + b1_ref[0]
    h = 0.5 * h * (1.0 + lax.erf(h * 0.7071067811865476))
    y = jnp.dot(h.astype(jnp.bfloat16), w2_ref[0].astype(jnp.bfloat16),
                preferred_element_type=jnp.float32)
    out_ref[...] = (y + b2_ref[0]) * sc_ref[...]


def _ffn(tile_expert, x_pad, W1, b1, W2, b2, score_pad):
    return pl.pallas_call(
        _ffn_body,
        grid_spec=pltpu.PrefetchScalarGridSpec(
            num_scalar_prefetch=1,
            grid=(G,),
            in_specs=[
                pl.BlockSpec((T, DIM), lambda g, te: (g, 0)),
                pl.BlockSpec((1, DIM, HID), lambda g, te: (te[g], 0, 0)),
                pl.BlockSpec((1, 1, HID), lambda g, te: (te[g], 0, 0)),
                pl.BlockSpec((1, HID, DIM), lambda g, te: (te[g], 0, 0)),
                pl.BlockSpec((1, 1, DIM), lambda g, te: (te[g], 0, 0)),
                pl.BlockSpec((T, 1), lambda g, te: (g, 0)),
            ],
            out_specs=pl.BlockSpec((T, DIM), lambda g, te: (g, 0)),
        ),
        out_shape=jax.ShapeDtypeStruct((ROWS, DIM), jnp.float32),
        compiler_params=pltpu.CompilerParams(
            dimension_semantics=("arbitrary",),
            vmem_limit_bytes=100 * 1024 * 1024,
        ),
    )(tile_expert, x_pad, W1, b1.reshape(E, 1, HID), W2,
      b2.reshape(E, 1, DIM), score_pad)


def kernel(x, gate_W, gate_b, W1, b1, W2, b2):
    x2 = x.reshape(N, DIM)
    idx, score = _gate(x2, gate_W, gate_b)

    oh = (idx[:, None] == jnp.arange(E, dtype=jnp.int32)[None, :]).astype(jnp.int32)
    counts = oh.sum(axis=0)
    rank = jnp.take_along_axis(jnp.cumsum(oh, axis=0), idx[:, None], axis=1)[:, 0] - 1
    ntiles = (counts + T - 1) // T
    tile_off = jnp.concatenate([jnp.zeros((1,), jnp.int32),
                                jnp.cumsum(ntiles).astype(jnp.int32)])
    dst = tile_off[idx] * T + rank
    src_pad = (jnp.arange(ROWS, dtype=jnp.int32) % N).at[dst].set(
        jnp.arange(N, dtype=jnp.int32))
    g_ids = jnp.arange(G, dtype=jnp.int32)
    te = jnp.minimum(
        jnp.searchsorted(tile_off[1:], g_ids, side="right").astype(jnp.int32),
        E - 1)

    x_pad = _dispatch(x2, src_pad)
    score_pad = jnp.zeros((ROWS,), jnp.float32).at[dst].set(score)

    y_pad = _ffn(te, x_pad, W1, b1, W2, b2, score_pad.reshape(ROWS, 1))

    out = _combine(y_pad, dst)
    return out.reshape(1, N, DIM)


# speedup vs baseline: 1.6595x; 1.0485x over previous
import functools

import jax
import jax.numpy as jnp
from jax import lax
from jax.experimental import pallas as pl
from jax.experimental.pallas import tpu as pltpu
from jax.experimental.pallas import tpu_sc as plsc

DIM = 1024
E = 8
HID = 2048
N = 2048
EP = 128
T = 256
G = 16
ROWS = G * T

NC = 2
NS = 16
NW = NC * NS
L = 16


def _gate_body(x_ref, w_ref, b_ref, idx_ref, val_ref, rank_ref, cnt_ref,
               carry_ref):
    g = pl.program_id(0)
    s = jnp.dot(x_ref[...], w_ref[...], preferred_element_type=jnp.float32)
    s = s + b_ref[...]
    m = jnp.max(s, axis=-1, keepdims=True)
    lane = lax.broadcasted_iota(jnp.int32, s.shape, 1)
    cand = jnp.where(s >= m, lane, EP)
    idxv = jnp.min(cand, axis=-1, keepdims=True)
    idx_ref[...] = idxv
    val_ref[...] = m

    @pl.when(g == 0)
    def _():
        carry_ref[...] = jnp.zeros_like(carry_ref)

    ohf = (lane == idxv).astype(jnp.float32)
    bt = ohf.shape[0]
    ri = lax.broadcasted_iota(jnp.int32, (bt, bt), 0)
    cj = lax.broadcasted_iota(jnp.int32, (bt, bt), 1)
    tri = (cj < ri).astype(jnp.float32)
    cum_excl = jnp.dot(tri, ohf, preferred_element_type=jnp.float32)
    carry = carry_ref[...]
    rank_ref[...] = jnp.sum(ohf * (cum_excl + carry), axis=-1,
                            keepdims=True).astype(jnp.int32)
    total = carry + jnp.sum(ohf, axis=0, keepdims=True)
    carry_ref[...] = total
    cnt_ref[...] = total.astype(jnp.int32)


def _gate(x2, gate_W, gate_b):
    wp = jnp.zeros((DIM, EP), jnp.float32).at[:, :E].set(gate_W)
    bp = jnp.full((1, EP), -1e30, jnp.float32).at[0, :E].set(gate_b)
    bt = 256
    idx, val, rank, cnt = pl.pallas_call(
        _gate_body,
        grid=(N // bt,),
        in_specs=[
            pl.BlockSpec((bt, DIM), lambda g: (g, 0)),
            pl.BlockSpec((DIM, EP), lambda g: (0, 0)),
            pl.BlockSpec((1, EP), lambda g: (0, 0)),
        ],
        out_specs=[
            pl.BlockSpec((bt, 1), lambda g: (g, 0)),
            pl.BlockSpec((bt, 1), lambda g: (g, 0)),
            pl.BlockSpec((bt, 1), lambda g: (g, 0)),
            pl.BlockSpec((1, EP), lambda g: (0, 0)),
        ],
        out_shape=[
            jax.ShapeDtypeStruct((N, 1), jnp.int32),
            jax.ShapeDtypeStruct((N, 1), jnp.float32),
            jax.ShapeDtypeStruct((N, 1), jnp.int32),
            jax.ShapeDtypeStruct((1, EP), jnp.int32),
        ],
        scratch_shapes=[pltpu.VMEM((1, EP), jnp.float32)],
        compiler_params=pltpu.CompilerParams(
            dimension_semantics=("arbitrary",),
        ),
    )(x2, wp, bp)
    return idx.reshape(N), val.reshape(N), rank.reshape(N), cnt[0, :E]


def _row_gather_body(tab_hbm, ind_hbm, out_hbm, idx_v, buf0, buf1, sem0, sem1,
                     *, nrows):
    wid = lax.axis_index("s") * NC + lax.axis_index("c")
    rpw = nrows // NW
    base = wid * rpw
    nchunk = rpw // 32
    for c in range(nchunk):
        pltpu.sync_copy(ind_hbm.at[pl.ds(base + 32 * c, 32)], idx_v.at[c])
    bufs = (buf0, buf1)
    sems = (sem0, sem1)
    cps = [None, None]
    cps[0] = pltpu.async_copy(tab_hbm.at[idx_v.at[0]], bufs[0], sems[0])
    for c in range(nchunk):
        if c + 1 < nchunk:
            cps[(c + 1) % 2] = pltpu.async_copy(
                tab_hbm.at[idx_v.at[c + 1]], bufs[(c + 1) % 2],
                sems[(c + 1) % 2])
        cps[c % 2].wait()
        pltpu.sync_copy(bufs[c % 2], out_hbm.at[pl.ds(base + 32 * c, 32)])


@functools.lru_cache(maxsize=None)
def _row_gather(nrows):
    return pl.kernel(
        functools.partial(_row_gather_body, nrows=nrows),
        out_type=jax.ShapeDtypeStruct((nrows, DIM), jnp.float32),
        mesh=plsc.VectorSubcoreMesh(core_axis_name="c", subcore_axis_name="s"),
        scratch_types=[
            pltpu.VMEM((nrows // NW // 32, 32), jnp.int32),
            pltpu.VMEM((32, DIM), jnp.float32),
            pltpu.VMEM((32, DIM), jnp.float32),
            pltpu.SemaphoreType.DMA,
            pltpu.SemaphoreType.DMA,
        ],
    )


def _ffn_body(te_ref, xp_ref, w1_ref, b1_ref, w2_ref, b2_ref, sc_ref, out_ref):
    xb = xp_ref[...].astype(jnp.bfloat16)
    h = jnp.dot(xb, w1_ref[0].astype(jnp.bfloat16),
                preferred_element_type=jnp.float32)
    h = h + b1_ref[0]
    h = 0.5 * h * (1.0 + lax.erf(h * 0.7071067811865476))
    y = jnp.dot(h.astype(jnp.bfloat16), w2_ref[0].astype(jnp.bfloat16),
                preferred_element_type=jnp.float32)
    out_ref[...] = (y + b2_ref[0]) * sc_ref[...]


def _ffn(tile_expert, x_pad, W1, b1, W2, b2, score_pad):
    return pl.pallas_call(
        _ffn_body,
        grid_spec=pltpu.PrefetchScalarGridSpec(
            num_scalar_prefetch=1,
            grid=(G,),
            in_specs=[
                pl.BlockSpec((T, DIM), lambda g, te: (g, 0)),
                pl.BlockSpec((1, DIM, HID), lambda g, te: (te[g], 0, 0)),
                pl.BlockSpec((1, 1, HID), lambda g, te: (te[g], 0, 0)),
                pl.BlockSpec((1, HID, DIM), lambda g, te: (te[g], 0, 0)),
                pl.BlockSpec((1, 1, DIM), lambda g, te: (te[g], 0, 0)),
                pl.BlockSpec((T, 1), lambda g, te: (g, 0)),
            ],
            out_specs=pl.BlockSpec((T, DIM), lambda g, te: (g, 0)),
        ),
        out_shape=jax.ShapeDtypeStruct((ROWS, DIM), jnp.float32),
        compiler_params=pltpu.CompilerParams(
            dimension_semantics=("arbitrary",),
            vmem_limit_bytes=100 * 1024 * 1024,
        ),
    )(tile_expert, x_pad, W1, b1.reshape(E, 1, HID), W2,
      b2.reshape(E, 1, DIM), score_pad)


def kernel(x, gate_W, gate_b, W1, b1, W2, b2):
    x2 = x.reshape(N, DIM)
    idx, score, rank, counts = _gate(x2, gate_W, gate_b)

    ntiles = (counts + T - 1) // T
    tile_off = jnp.concatenate([jnp.zeros((1,), jnp.int32),
                                jnp.cumsum(ntiles).astype(jnp.int32)])
    dst = tile_off[idx] * T + rank
    src_pad = (jnp.arange(ROWS, dtype=jnp.int32) % N).at[dst].set(
        jnp.arange(N, dtype=jnp.int32))
    g_ids = jnp.arange(G, dtype=jnp.int32)
    te = jnp.minimum(
        jnp.searchsorted(tile_off[1:], g_ids, side="right").astype(jnp.int32),
        E - 1)

    x_pad = _row_gather(ROWS)(x2, src_pad)
    score_pad = jnp.zeros((ROWS,), jnp.float32).at[dst].set(score)

    y_pad = _ffn(te, x_pad, W1, b1, W2, b2, score_pad.reshape(ROWS, 1))

    out = _row_gather(N)(y_pad, dst)
    return out.reshape(1, N, DIM)


# speedup vs baseline: 1.7004x; 1.0246x over previous
import functools

import jax
import jax.numpy as jnp
from jax import lax
from jax.experimental import pallas as pl
from jax.experimental.pallas import tpu as pltpu
from jax.experimental.pallas import tpu_sc as plsc

DIM = 1024
E = 8
HID = 2048
N = 2048
EP = 128
T = 256
G = 16
ROWS = G * T

NC = 2
NS = 16
NW = NC * NS
L = 16


def _gate_body(x_ref, w_ref, b_ref, idx_ref, val_ref, rank_ref, cnt_ref,
               carry_ref):
    g = pl.program_id(0)
    s = jnp.dot(x_ref[...], w_ref[...], preferred_element_type=jnp.float32)
    s = s + b_ref[...]
    m = jnp.max(s, axis=-1, keepdims=True)
    lane = lax.broadcasted_iota(jnp.int32, s.shape, 1)
    cand = jnp.where(s >= m, lane, EP)
    idxv = jnp.min(cand, axis=-1, keepdims=True)
    idx_ref[...] = idxv
    val_ref[...] = m

    @pl.when(g == 0)
    def _():
        carry_ref[...] = jnp.zeros_like(carry_ref)

    ohf = (lane == idxv).astype(jnp.float32)
    bt = ohf.shape[0]
    ri = lax.broadcasted_iota(jnp.int32, (bt, bt), 0)
    cj = lax.broadcasted_iota(jnp.int32, (bt, bt), 1)
    tri = (cj < ri).astype(jnp.float32)
    cum_excl = jnp.dot(tri, ohf, preferred_element_type=jnp.float32)
    carry = carry_ref[...]
    rank_ref[...] = jnp.sum(ohf * (cum_excl + carry), axis=-1,
                            keepdims=True).astype(jnp.int32)
    total = carry + jnp.sum(ohf, axis=0, keepdims=True)
    carry_ref[...] = total
    cnt_ref[...] = total.astype(jnp.int32)


def _gate(x2, gate_W, gate_b):
    wp = jnp.zeros((DIM, EP), jnp.float32).at[:, :E].set(gate_W)
    bp = jnp.full((1, EP), -1e30, jnp.float32).at[0, :E].set(gate_b)
    bt = 256
    idx, val, rank, cnt = pl.pallas_call(
        _gate_body,
        grid=(N // bt,),
        in_specs=[
            pl.BlockSpec((bt, DIM), lambda g: (g, 0)),
            pl.BlockSpec((DIM, EP), lambda g: (0, 0)),
            pl.BlockSpec((1, EP), lambda g: (0, 0)),
        ],
        out_specs=[
            pl.BlockSpec((bt, 1), lambda g: (g, 0)),
            pl.BlockSpec((bt, 1), lambda g: (g, 0)),
            pl.BlockSpec((bt, 1), lambda g: (g, 0)),
            pl.BlockSpec((1, EP), lambda g: (0, 0)),
        ],
        out_shape=[
            jax.ShapeDtypeStruct((N, 1), jnp.int32),
            jax.ShapeDtypeStruct((N, 1), jnp.float32),
            jax.ShapeDtypeStruct((N, 1), jnp.int32),
            jax.ShapeDtypeStruct((1, EP), jnp.int32),
        ],
        scratch_shapes=[pltpu.VMEM((1, EP), jnp.float32)],
        compiler_params=pltpu.CompilerParams(
            dimension_semantics=("arbitrary",),
        ),
    )(x2, wp, bp)
    return idx.reshape(N), val.reshape(N), rank.reshape(N), cnt[0, :E]


def _row_gather_body(tab_hbm, ind_hbm, out_hbm, idx_v, buf0, buf1, sem0, sem1,
                     *, nrows):
    wid = lax.axis_index("s") * NC + lax.axis_index("c")
    rpw = nrows // NW
    base = wid * rpw
    nchunk = rpw // 32
    for c in range(nchunk):
        pltpu.sync_copy(ind_hbm.at[pl.ds(base + 32 * c, 32)], idx_v.at[c])
    bufs = (buf0, buf1)
    sems = (sem0, sem1)
    cps = [None, None]
    cps[0] = pltpu.async_copy(tab_hbm.at[idx_v.at[0]], bufs[0], sems[0])
    for c in range(nchunk):
        if c + 1 < nchunk:
            cps[(c + 1) % 2] = pltpu.async_copy(
                tab_hbm.at[idx_v.at[c + 1]], bufs[(c + 1) % 2],
                sems[(c + 1) % 2])
        cps[c % 2].wait()
        pltpu.sync_copy(bufs[c % 2], out_hbm.at[pl.ds(base + 32 * c, 32)])


@functools.lru_cache(maxsize=None)
def _row_gather(nrows):
    return pl.kernel(
        functools.partial(_row_gather_body, nrows=nrows),
        out_type=jax.ShapeDtypeStruct((nrows, DIM), jnp.float32),
        mesh=plsc.VectorSubcoreMesh(core_axis_name="c", subcore_axis_name="s"),
        scratch_types=[
            pltpu.VMEM((nrows // NW // 32, 32), jnp.int32),
            pltpu.VMEM((32, DIM), jnp.float32),
            pltpu.VMEM((32, DIM), jnp.float32),
            pltpu.SemaphoreType.DMA,
            pltpu.SemaphoreType.DMA,
        ],
    )


NCH = 4
CH = HID // NCH


def _ffn_body(nt_ref, ts_ref, xp_ref, w1_any, b1_ref, w2_any, b2_ref, sc_ref,
              out_ref, w1ring, w2ring, w1sem, w2sem):
    e = pl.program_id(0)

    def w1_copy(ee, c):
        return pltpu.make_async_copy(
            w1_any.at[ee, :, pl.ds(c * CH, CH)], w1ring.at[c], w1sem.at[c])

    def w2_copy(ee, c):
        return pltpu.make_async_copy(
            w2_any.at[ee, pl.ds(c * CH, CH), :], w2ring.at[c], w2sem.at[c])

    @pl.when(e == 0)
    def _():
        for c in range(NCH):
            w1_copy(0, c).start()
            w2_copy(0, c).start()

    for c in range(NCH):
        w1_copy(e, c).wait()
        w2_copy(e, c).wait()
        w1b = w1ring[c].astype(jnp.bfloat16)
        w2b = w2ring[c].astype(jnp.bfloat16)
        b1row = b1_ref[pl.ds(e, 1), 0, pl.ds(c * CH, CH)]
        nt = nt_ref[e]
        ts = ts_ref[e]

        def tile_body(j, _):
            row0 = (ts + j) * T
            xb = xp_ref[pl.ds(row0, T), :].astype(jnp.bfloat16)
            h = jnp.dot(xb, w1b, preferred_element_type=jnp.float32) + b1row
            h = 0.5 * h * (1.0 + lax.erf(h * 0.7071067811865476))
            yp = jnp.dot(h.astype(jnp.bfloat16), w2b,
                         preferred_element_type=jnp.float32)
            if c == 0:
                out_ref[pl.ds(row0, T), :] = yp
            elif c == NCH - 1:
                b2row = b2_ref[pl.ds(e, 1), 0, :]
                acc = out_ref[pl.ds(row0, T), :] + yp + b2row
                out_ref[pl.ds(row0, T), :] = acc * sc_ref[pl.ds(row0, T), :]
            else:
                out_ref[pl.ds(row0, T), :] = out_ref[pl.ds(row0, T), :] + yp
            return 0

        lax.fori_loop(0, nt, tile_body, 0)

        @pl.when(e + 1 < E)
        def _():
            w1_copy(e + 1, c).start()
            w2_copy(e + 1, c).start()


def _ffn(ntiles, tstart, x_pad, W1, b1, W2, b2, score_pad):
    return pl.pallas_call(
        _ffn_body,
        grid_spec=pltpu.PrefetchScalarGridSpec(
            num_scalar_prefetch=2,
            grid=(E,),
            in_specs=[
                pl.BlockSpec((ROWS, DIM), lambda e, nt, ts: (0, 0)),
                pl.BlockSpec(memory_space=pl.ANY),
                pl.BlockSpec((E, 1, HID), lambda e, nt, ts: (0, 0, 0)),
                pl.BlockSpec(memory_space=pl.ANY),
                pl.BlockSpec((E, 1, DIM), lambda e, nt, ts: (0, 0, 0)),
                pl.BlockSpec((ROWS, 1), lambda e, nt, ts: (0, 0)),
            ],
            out_specs=pl.BlockSpec((ROWS, DIM), lambda e, nt, ts: (0, 0)),
            scratch_shapes=[
                pltpu.VMEM((NCH, DIM, CH), jnp.float32),
                pltpu.VMEM((NCH, CH, DIM), jnp.float32),
                pltpu.SemaphoreType.DMA((NCH,)),
                pltpu.SemaphoreType.DMA((NCH,)),
            ],
        ),
        out_shape=jax.ShapeDtypeStruct((ROWS, DIM), jnp.float32),
        compiler_params=pltpu.CompilerParams(
            dimension_semantics=("arbitrary",),
            vmem_limit_bytes=100 * 1024 * 1024,
        ),
    )(ntiles, tstart, x_pad, W1, b1.reshape(E, 1, HID), W2,
      b2.reshape(E, 1, DIM), score_pad)


def kernel(x, gate_W, gate_b, W1, b1, W2, b2):
    x2 = x.reshape(N, DIM)
    idx, score, rank, counts = _gate(x2, gate_W, gate_b)

    ntiles = (counts + T - 1) // T
    tile_off = jnp.concatenate([jnp.zeros((1,), jnp.int32),
                                jnp.cumsum(ntiles).astype(jnp.int32)])
    dst = tile_off[idx] * T + rank
    src_pad = (jnp.arange(ROWS, dtype=jnp.int32) % N).at[dst].set(
        jnp.arange(N, dtype=jnp.int32))

    x_pad = _row_gather(ROWS)(x2, src_pad)
    score_pad = jnp.zeros((ROWS,), jnp.float32).at[dst].set(score)

    y_pad = _ffn(ntiles, tile_off[:E], x_pad, W1, b1, W2, b2,
                 score_pad.reshape(ROWS, 1))

    out = _row_gather(N)(y_pad, dst)
    return out.reshape(1, N, DIM)


# speedup vs baseline: 1.7090x; 1.0050x over previous
import functools

import jax
import jax.numpy as jnp
from jax import lax
from jax.experimental import pallas as pl
from jax.experimental.pallas import tpu as pltpu
from jax.experimental.pallas import tpu_sc as plsc

DIM = 1024
E = 8
HID = 2048
N = 2048
EP = 128
T = 256
G = 16
ROWS = G * T

NC = 2
NS = 16
NW = NC * NS
L = 16


def _gate_body(x_ref, w_ref, b_ref, idx_ref, val_ref, rank_ref, cnt_ref,
               carry_ref):
    g = pl.program_id(0)
    s = jnp.dot(x_ref[...], w_ref[...], preferred_element_type=jnp.float32)
    s = s + b_ref[...]
    m = jnp.max(s, axis=-1, keepdims=True)
    lane = lax.broadcasted_iota(jnp.int32, s.shape, 1)
    cand = jnp.where(s >= m, lane, EP)
    idxv = jnp.min(cand, axis=-1, keepdims=True)
    idx_ref[...] = idxv
    val_ref[...] = m

    @pl.when(g == 0)
    def _():
        carry_ref[...] = jnp.zeros_like(carry_ref)

    ohf = (lane == idxv).astype(jnp.float32)
    bt = ohf.shape[0]
    ri = lax.broadcasted_iota(jnp.int32, (bt, bt), 0)
    cj = lax.broadcasted_iota(jnp.int32, (bt, bt), 1)
    tri = (cj < ri).astype(jnp.float32)
    cum_excl = jnp.dot(tri, ohf, preferred_element_type=jnp.float32)
    carry = carry_ref[...]
    rank_ref[...] = jnp.sum(ohf * (cum_excl + carry), axis=-1,
                            keepdims=True).astype(jnp.int32)
    total = carry + jnp.sum(ohf, axis=0, keepdims=True)
    carry_ref[...] = total
    cnt_ref[...] = total.astype(jnp.int32)


def _gate(x2, gate_W, gate_b):
    wp = jnp.zeros((DIM, EP), jnp.float32).at[:, :E].set(gate_W)
    bp = jnp.full((1, EP), -1e30, jnp.float32).at[0, :E].set(gate_b)
    bt = 256
    idx, val, rank, cnt = pl.pallas_call(
        _gate_body,
        grid=(N // bt,),
        in_specs=[
            pl.BlockSpec((bt, DIM), lambda g: (g, 0)),
            pl.BlockSpec((DIM, EP), lambda g: (0, 0)),
            pl.BlockSpec((1, EP), lambda g: (0, 0)),
        ],
        out_specs=[
            pl.BlockSpec((bt, 1), lambda g: (g, 0)),
            pl.BlockSpec((bt, 1), lambda g: (g, 0)),
            pl.BlockSpec((bt, 1), lambda g: (g, 0)),
            pl.BlockSpec((1, EP), lambda g: (0, 0)),
        ],
        out_shape=[
            jax.ShapeDtypeStruct((N, 1), jnp.int32),
            jax.ShapeDtypeStruct((N, 1), jnp.float32),
            jax.ShapeDtypeStruct((N, 1), jnp.int32),
            jax.ShapeDtypeStruct((1, EP), jnp.int32),
        ],
        scratch_shapes=[pltpu.VMEM((1, EP), jnp.float32)],
        compiler_params=pltpu.CompilerParams(
            dimension_semantics=("arbitrary",),
        ),
    )(x2, wp, bp)
    return idx.reshape(N), val.reshape(N), rank.reshape(N), cnt[0, :E]


def _row_gather_body(tab_hbm, ind_hbm, out_hbm, idx_v, buf0, buf1,
                     sem0, sem1, wsem0, wsem1, *, nrows):
    wid = lax.axis_index("s") * NC + lax.axis_index("c")
    rpw = nrows // NW
    base = wid * rpw
    nchunk = rpw // 32
    pltpu.sync_copy(ind_hbm.at[pl.ds(wid * nchunk, nchunk)], idx_v)
    bufs = (buf0, buf1)
    sems = (sem0, sem1)
    wsems = (wsem0, wsem1)

    def wcopy(c):
        return pltpu.make_async_copy(
            bufs[c % 2], out_hbm.at[pl.ds(base + 32 * c, 32)], wsems[c % 2])

    cps = [None, None]
    cps[0] = pltpu.async_copy(tab_hbm.at[idx_v.at[0]], bufs[0], sems[0])
    for c in range(nchunk):
        if c + 1 < nchunk:
            if c >= 1:
                wcopy(c - 1).wait()
            cps[(c + 1) % 2] = pltpu.async_copy(
                tab_hbm.at[idx_v.at[c + 1]], bufs[(c + 1) % 2],
                sems[(c + 1) % 2])
        cps[c % 2].wait()
        wcopy(c).start()
    for c in range(max(nchunk - 2, 0), nchunk):
        wcopy(c).wait()


@functools.lru_cache(maxsize=None)
def _row_gather(nrows):
    return pl.kernel(
        functools.partial(_row_gather_body, nrows=nrows),
        out_type=jax.ShapeDtypeStruct((nrows, DIM), jnp.float32),
        mesh=plsc.VectorSubcoreMesh(core_axis_name="c", subcore_axis_name="s"),
        scratch_types=[
            pltpu.VMEM((nrows // NW // 32, 32), jnp.int32),
            pltpu.VMEM((32, DIM), jnp.float32),
            pltpu.VMEM((32, DIM), jnp.float32),
            pltpu.SemaphoreType.DMA,
            pltpu.SemaphoreType.DMA,
            pltpu.SemaphoreType.DMA,
            pltpu.SemaphoreType.DMA,
        ],
    )


NCH = 4
CH = HID // NCH


def _ffn_body(nt_ref, ts_ref, xp_ref, w1_any, b1_ref, w2_any, b2_ref, sc_ref,
              out_ref, w1ring, w2ring, w1sem, w2sem):
    e = pl.program_id(0)

    def w1_copy(ee, c):
        return pltpu.make_async_copy(
            w1_any.at[ee, :, pl.ds(c * CH, CH)], w1ring.at[c], w1sem.at[c])

    def w2_copy(ee, c):
        return pltpu.make_async_copy(
            w2_any.at[ee, pl.ds(c * CH, CH), :], w2ring.at[c], w2sem.at[c])

    @pl.when(e == 0)
    def _():
        for c in range(NCH):
            w1_copy(0, c).start()
            w2_copy(0, c).start()

    for c in range(NCH):
        w1_copy(e, c).wait()
        w2_copy(e, c).wait()
        w1b = w1ring[c].astype(jnp.bfloat16)
        w2b = w2ring[c].astype(jnp.bfloat16)
        b1row = b1_ref[pl.ds(e, 1), 0, pl.ds(c * CH, CH)]
        nt = nt_ref[e]
        ts = ts_ref[e]

        def tile_body(j, _):
            row0 = (ts + j) * T
            xb = xp_ref[pl.ds(row0, T), :].astype(jnp.bfloat16)
            h = jnp.dot(xb, w1b, preferred_element_type=jnp.float32) + b1row
            h = 0.5 * h * (1.0 + lax.erf(h * 0.7071067811865476))
            yp = jnp.dot(h.astype(jnp.bfloat16), w2b,
                         preferred_element_type=jnp.float32)
            if c == 0:
                out_ref[pl.ds(row0, T), :] = yp
            elif c == NCH - 1:
                b2row = b2_ref[pl.ds(e, 1), 0, :]
                acc = out_ref[pl.ds(row0, T), :] + yp + b2row
                out_ref[pl.ds(row0, T), :] = acc * sc_ref[pl.ds(row0, T), :]
            else:
                out_ref[pl.ds(row0, T), :] = out_ref[pl.ds(row0, T), :] + yp
            return 0

        lax.fori_loop(0, nt, tile_body, 0)

        @pl.when(e + 1 < E)
        def _():
            w1_copy(e + 1, c).start()
            w2_copy(e + 1, c).start()


def _ffn(ntiles, tstart, x_pad, W1, b1, W2, b2, score_pad):
    return pl.pallas_call(
        _ffn_body,
        grid_spec=pltpu.PrefetchScalarGridSpec(
            num_scalar_prefetch=2,
            grid=(E,),
            in_specs=[
                pl.BlockSpec((ROWS, DIM), lambda e, nt, ts: (0, 0)),
                pl.BlockSpec(memory_space=pl.ANY),
                pl.BlockSpec((E, 1, HID), lambda e, nt, ts: (0, 0, 0)),
                pl.BlockSpec(memory_space=pl.ANY),
                pl.BlockSpec((E, 1, DIM), lambda e, nt, ts: (0, 0, 0)),
                pl.BlockSpec((ROWS, 1), lambda e, nt, ts: (0, 0)),
            ],
            out_specs=pl.BlockSpec((ROWS, DIM), lambda e, nt, ts: (0, 0)),
            scratch_shapes=[
                pltpu.VMEM((NCH, DIM, CH), jnp.float32),
                pltpu.VMEM((NCH, CH, DIM), jnp.float32),
                pltpu.SemaphoreType.DMA((NCH,)),
                pltpu.SemaphoreType.DMA((NCH,)),
            ],
        ),
        out_shape=jax.ShapeDtypeStruct((ROWS, DIM), jnp.float32),
        compiler_params=pltpu.CompilerParams(
            dimension_semantics=("arbitrary",),
            vmem_limit_bytes=100 * 1024 * 1024,
        ),
    )(ntiles, tstart, x_pad, W1, b1.reshape(E, 1, HID), W2,
      b2.reshape(E, 1, DIM), score_pad)


def kernel(x, gate_W, gate_b, W1, b1, W2, b2):
    x2 = x.reshape(N, DIM)
    idx, score, rank, counts = _gate(x2, gate_W, gate_b)

    ntiles = (counts + T - 1) // T
    tile_off = jnp.concatenate([jnp.zeros((1,), jnp.int32),
                                jnp.cumsum(ntiles).astype(jnp.int32)])
    dst = tile_off[idx] * T + rank
    src_pad = (jnp.arange(ROWS, dtype=jnp.int32) % N).at[dst].set(
        jnp.arange(N, dtype=jnp.int32))

    x_pad = _row_gather(ROWS)(x2, src_pad.reshape(ROWS // 32, 32))
    score_pad = jnp.zeros((ROWS,), jnp.float32).at[dst].set(score)

    y_pad = _ffn(ntiles, tile_off[:E], x_pad, W1, b1, W2, b2,
                 score_pad.reshape(ROWS, 1))

    out = _row_gather(N)(y_pad, dst.reshape(N // 32, 32))
    return out.reshape(1, N, DIM)
